# baseline (speedup 1.0000x reference)
import functools

import jax
import jax.numpy as jnp
from jax import lax
from jax.experimental import pallas as pl
from jax.experimental.pallas import tpu as pltpu
from jax.experimental.pallas import tpu_sc as plsc

EPS = 1e-5

NC, NS = 2, 16
NW = NC * NS
NBUF = 4
TPAD = 137


def _normalize_table(table, w, b):
    V, D = table.shape
    RB = 2000
    assert V % RB == 0

    def body(t_ref, w_ref, b_ref, o_ref):
        e = t_ref[...]
        mu = jnp.mean(e, axis=-1, keepdims=True)
        d = e - mu
        var = jnp.mean(d * d, axis=-1, keepdims=True)
        o_ref[...] = d * lax.rsqrt(var + EPS) * w_ref[...] + b_ref[...]

    return pl.pallas_call(
        body,
        grid=(V // RB,),
        in_specs=[
            pl.BlockSpec((RB, D), lambda i: (i, 0)),
            pl.BlockSpec((1, D), lambda i: (0, 0)),
            pl.BlockSpec((1, D), lambda i: (0, 0)),
        ],
        out_specs=pl.BlockSpec((RB, D), lambda i: (i, 0)),
        out_shape=jax.ShapeDtypeStruct((V, D), jnp.float32),
    )(table, w.reshape(1, D), b.reshape(1, D))


def _gather_mask_sc(ntable, x):
    V, D = ntable.shape
    B, S = x.shape
    BW = B // NW
    assert BW * NW == B and BW == 128 and D == 64 and S % NBUF == 0

    mesh = plsc.VectorSubcoreMesh(
        core_axis_name="c", subcore_axis_name="s",
        num_cores=NC, num_subcores=NS)

    @functools.partial(
        pl.kernel,
        out_type=[
            jax.ShapeDtypeStruct((S * D, B), jnp.float32),
            jax.ShapeDtypeStruct((B, S), jnp.int32),
        ],
        mesh=mesh,
        compiler_params=pltpu.CompilerParams(
            use_tc_tiling_on_sc=False, needs_layout_passes=False),
        scratch_types=[
            pltpu.VMEM((BW, S), jnp.int32),
            pltpu.VMEM((S, BW), jnp.int32),
            pltpu.VMEM((NBUF, BW, D), jnp.float32),
            pltpu.VMEM((NBUF, D, TPAD), jnp.float32),
        ]
        + [pltpu.SemaphoreType.DMA] * (2 * NBUF),
    )
    def k(tab_hbm, x_hbm, out_hbm, mask_hbm, idx_v, idx_t, gbuf, tbuf,
          *sems):
        gat_sems = sems[:NBUF]
        out_sems = sems[NBUF:]
        wid = lax.axis_index("s") * NC + lax.axis_index("c")
        b0 = wid * BW

        pltpu.sync_copy(x_hbm.at[pl.ds(b0, BW)], idx_v)

        def tr_idx(brel, carry):
            iota16 = lax.iota(jnp.int32, 16)
            col = jnp.full((16,), brel, jnp.int32)
            for l in range(S // 16):
                v = idx_v[brel, pl.ds(16 * l, 16)]
                plsc.store_scatter(idx_t, [iota16 + 16 * l, col], v)
            if S % 16:
                v = idx_v[brel, pl.ds(S - 16, 16)]
                plsc.store_scatter(
                    idx_t, [iota16 + (S - 16), col], v)
            return carry

        lax.fori_loop(0, BW, tr_idx, 0, unroll=False)

        def mask_row(r, carry):
            for off in range(0, S - 15, 16):
                off = min(off, S - 16)
                iv = idx_v[r, pl.ds(off, 16)]
                idx_v[r, pl.ds(off, 16)] = jnp.where(
                    iv > 0, jnp.int32(1), jnp.int32(0))
            if S % 16:
                iv = idx_v[r, pl.ds(S - 16, 16)]
                idx_v[r, pl.ds(S - 16, 16)] = jnp.where(
                    iv > 0, jnp.int32(1), jnp.int32(0))
            return carry

        lax.fori_loop(0, BW, mask_row, 0, unroll=False)
        pltpu.sync_copy(idx_v, mask_hbm.at[pl.ds(b0, BW)])

        def fire_gather(s, b):
            pltpu.async_copy(
                tab_hbm.at[idx_t.at[s]], gbuf.at[b], gat_sems[b])

        def wait_gather(b):
            pltpu.make_async_copy(
                tab_hbm.at[idx_t.at[0]], gbuf.at[b], gat_sems[b]).wait()

        def out_dst(s):
            return out_hbm.at[pl.ds(s * D, D), pl.ds(b0, BW)]

        def fire_out(s, b):
            pltpu.async_copy(
                tbuf.at[b].at[:, pl.ds(0, BW)], out_dst(s), out_sems[b])

        def wait_out(b):
            pltpu.make_async_copy(
                tbuf.at[b].at[:, pl.ds(0, BW)], out_dst(0),
                out_sems[b]).wait()

        def transpose_tile(b):
            def tr4(r4, carry):
                iota16 = lax.iota(jnp.int32, 16)
                for rr in range(4):
                    brel = r4 * 4 + rr
                    col = jnp.full((16,), brel, jnp.int32)
                    for kk in range(D // 16):
                        v = gbuf[b, brel, pl.ds(16 * kk, 16)]
                        plsc.store_scatter(
                            tbuf.at[b], [iota16 + 16 * kk, col], v)
                return carry

            lax.fori_loop(0, BW // 4, tr4, 0, unroll=False)

        def round_body(i, carry):
            for b in range(NBUF):
                g = i * NBUF + b
                pb = (b - 1) % NBUF
                fire_gather(g, b)

                @pl.when(g >= 5)
                def _(pb=pb):
                    wait_out(pb)

                @pl.when(g >= 1)
                def _(g=g, b=b, pb=pb):
                    wait_gather(pb)
                    transpose_tile(pb)
                    fire_out(g - 1, pb)
            return carry

        lax.fori_loop(0, S // NBUF, round_body, 0, unroll=False)

        lb = NBUF - 1
        wait_gather(lb)
        wait_out(lb)
        transpose_tile(lb)
        fire_out(S - 1, lb)
        for b in range(NBUF):
            wait_out(b)

    return k(ntable, x)


def kernel(x, table, ln_weight, ln_bias):
    B, S = x.shape
    V, D = table.shape
    assert B % NW == 0

    ntable = _normalize_table(table, ln_weight, ln_bias)
    out2d, mask_i32 = _gather_mask_sc(ntable, x.astype(jnp.int32))
    normed = out2d.reshape(S, D, B).transpose(2, 0, 1)
    mask = mask_i32 != 0
    return (normed, mask)


# speedup vs baseline: 1.0423x; 1.0423x over previous
import functools

import jax
import jax.numpy as jnp
from jax import lax
from jax.experimental import pallas as pl
from jax.experimental.pallas import tpu as pltpu
from jax.experimental.pallas import tpu_sc as plsc

EPS = 1e-5

NC, NS = 2, 16
NW = NC * NS
NBUF = 4
TPAD = 137


def _normalize_table(table, w, b):
    V, D = table.shape
    RB = 2000
    assert V % RB == 0

    def body(t_ref, w_ref, b_ref, o_ref):
        e = t_ref[...]
        mu = jnp.mean(e, axis=-1, keepdims=True)
        d = e - mu
        var = jnp.mean(d * d, axis=-1, keepdims=True)
        o_ref[...] = d * lax.rsqrt(var + EPS) * w_ref[...] + b_ref[...]

    return pl.pallas_call(
        body,
        grid=(V // RB,),
        in_specs=[
            pl.BlockSpec((RB, D), lambda i: (i, 0)),
            pl.BlockSpec((1, D), lambda i: (0, 0)),
            pl.BlockSpec((1, D), lambda i: (0, 0)),
        ],
        out_specs=pl.BlockSpec((RB, D), lambda i: (i, 0)),
        out_shape=jax.ShapeDtypeStruct((V, D), jnp.float32),
    )(table, w.reshape(1, D), b.reshape(1, D))


def _gather_mask_sc(ntable, x):
    V, D = ntable.shape
    B, S = x.shape
    BW = B // NW
    assert BW * NW == B and BW == 128 and D == 64 and S % NBUF == 0

    mesh = plsc.VectorSubcoreMesh(
        core_axis_name="c", subcore_axis_name="s",
        num_cores=NC, num_subcores=NS)

    @functools.partial(
        pl.kernel,
        out_type=[
            jax.ShapeDtypeStruct((S * D, B), jnp.float32),
            jax.ShapeDtypeStruct((B, S), jnp.int32),
        ],
        mesh=mesh,
        compiler_params=pltpu.CompilerParams(
            use_tc_tiling_on_sc=False, needs_layout_passes=False),
        scratch_types=[
            pltpu.VMEM((BW, S), jnp.int32),
            pltpu.VMEM((S, BW), jnp.int32),
            pltpu.VMEM((NBUF, BW, D), jnp.float32),
            pltpu.VMEM((NBUF, D, TPAD), jnp.float32),
        ]
        + [pltpu.SemaphoreType.DMA] * (2 * NBUF),
    )
    def k(tab_hbm, x_hbm, out_hbm, mask_hbm, idx_v, idx_t, gbuf, tbuf,
          *sems):
        gat_sems = sems[:NBUF]
        out_sems = sems[NBUF:]
        wid = lax.axis_index("s") * NC + lax.axis_index("c")
        b0 = wid * BW

        pltpu.sync_copy(x_hbm.at[pl.ds(b0, BW)], idx_v)

        def tr_idx(brel, carry):
            iota16 = lax.iota(jnp.int32, 16)
            col = jnp.full((16,), brel, jnp.int32)
            for l in range(S // 16):
                v = idx_v[brel, pl.ds(16 * l, 16)]
                plsc.store_scatter(idx_t, [iota16 + 16 * l, col], v)
            if S % 16:
                v = idx_v[brel, pl.ds(S - 16, 16)]
                plsc.store_scatter(idx_t, [iota16 + (S - 16), col], v)
            return carry

        lax.fori_loop(0, BW, tr_idx, 0, unroll=False)

        def mask_row(r, carry):
            for off in range(0, S - 15, 16):
                iv = idx_v[r, pl.ds(off, 16)]
                idx_v[r, pl.ds(off, 16)] = jnp.where(
                    iv > 0, jnp.int32(1), jnp.int32(0))
            if S % 16:
                iv = idx_v[r, pl.ds(S - 16, 16)]
                idx_v[r, pl.ds(S - 16, 16)] = jnp.where(
                    iv > 0, jnp.int32(1), jnp.int32(0))
            return carry

        lax.fori_loop(0, BW, mask_row, 0, unroll=False)
        pltpu.sync_copy(idx_v, mask_hbm.at[pl.ds(b0, BW)])

        def fire_gather(s, b):
            pltpu.async_copy(
                tab_hbm.at[idx_t.at[s]], gbuf.at[b], gat_sems[b])

        def wait_gather(b):
            pltpu.make_async_copy(
                tab_hbm.at[idx_t.at[0]], gbuf.at[b], gat_sems[b]).wait()

        def out_dst(s):
            return out_hbm.at[pl.ds(s * D, D), pl.ds(b0, BW)]

        def fire_out(s, b):
            pltpu.async_copy(
                tbuf.at[b].at[:, pl.ds(0, BW)], out_dst(s), out_sems[b])

        def wait_out(b):
            pltpu.make_async_copy(
                tbuf.at[b].at[:, pl.ds(0, BW)], out_dst(0),
                out_sems[b]).wait()

        def transpose_tile(b):
            def tr16(r16, carry):
                iota16 = lax.iota(jnp.int32, 16)
                rows = [iota16 + 16 * kk for kk in range(D // 16)]
                base = r16 * 16
                prev = None
                for rr in range(16):
                    col = jnp.full((16,), base + rr, jnp.int32)
                    cur = [(kk, col, gbuf[b, base + rr, pl.ds(16 * kk, 16)])
                           for kk in range(D // 16)]
                    if prev is not None:
                        for kk, pcol, v in prev:
                            plsc.store_scatter(
                                tbuf.at[b], [rows[kk], pcol], v)
                    prev = cur
                for kk, pcol, v in prev:
                    plsc.store_scatter(tbuf.at[b], [rows[kk], pcol], v)
                return carry

            lax.fori_loop(0, BW // 16, tr16, 0, unroll=False)

        def round_body(i, carry):
            for b in range(NBUF):
                g = i * NBUF + b
                pb = (b - 1) % NBUF
                fire_gather(g, b)

                @pl.when(g >= 5)
                def _(pb=pb):
                    wait_out(pb)

                @pl.when(g >= 1)
                def _(g=g, pb=pb):
                    wait_gather(pb)
                    transpose_tile(pb)
                    fire_out(g - 1, pb)
            return carry

        lax.fori_loop(0, S // NBUF, round_body, 0, unroll=False)

        lb = NBUF - 1
        wait_gather(lb)
        wait_out(lb)
        transpose_tile(lb)
        fire_out(S - 1, lb)
        for b in range(NBUF):
            wait_out(b)

    return k(ntable, x)


def kernel(x, table, ln_weight, ln_bias):
    B, S = x.shape
    V, D = table.shape
    assert B % NW == 0

    ntable = _normalize_table(table, ln_weight, ln_bias)
    out2d, mask_i32 = _gather_mask_sc(ntable, x.astype(jnp.int32))
    normed = out2d.reshape(S, D, B).transpose(2, 0, 1)
    mask = mask_i32 != 0
    return (normed, mask)


# speedup vs baseline: 1.2097x; 1.1606x over previous
import functools

import jax
import jax.numpy as jnp
from jax import lax
from jax.experimental import pallas as pl
from jax.experimental.pallas import tpu as pltpu
from jax.experimental.pallas import tpu_sc as plsc

EPS = 1e-5

NC, NS = 2, 16
NW = NC * NS
GRP = 128
KG = 2
BLK = KG * GRP
NBUF = 4


def _normalize_table(table, w, b):
    V, D = table.shape
    RB = 2000
    assert V % RB == 0

    def body(t_ref, w_ref, b_ref, o_ref):
        e = t_ref[...]
        mu = jnp.mean(e, axis=-1, keepdims=True)
        d = e - mu
        var = jnp.mean(d * d, axis=-1, keepdims=True)
        o_ref[...] = d * lax.rsqrt(var + EPS) * w_ref[...] + b_ref[...]

    return pl.pallas_call(
        body,
        grid=(V // RB,),
        in_specs=[
            pl.BlockSpec((RB, D), lambda i: (i, 0)),
            pl.BlockSpec((1, D), lambda i: (0, 0)),
            pl.BlockSpec((1, D), lambda i: (0, 0)),
        ],
        out_specs=pl.BlockSpec((RB, D), lambda i: (i, 0)),
        out_shape=jax.ShapeDtypeStruct((V, D), jnp.float32),
    )(table, w.reshape(1, D), b.reshape(1, D))


def _gather_mask_sc(ntable, x2d):
    V, D = ntable.shape
    NR, _ = x2d.shape
    N = NR * GRP
    per_w = N // NW
    rows_w = per_w // GRP
    nblk = per_w // BLK
    assert per_w * NW == N and nblk * BLK == per_w and nblk % NBUF == 0

    mesh = plsc.VectorSubcoreMesh(
        core_axis_name="c", subcore_axis_name="s",
        num_cores=NC, num_subcores=NS)

    @functools.partial(
        pl.kernel,
        out_type=[
            jax.ShapeDtypeStruct((NR, GRP, D), jnp.float32),
            jax.ShapeDtypeStruct((NR, GRP), jnp.int32),
        ],
        mesh=mesh,
        compiler_params=pltpu.CompilerParams(use_tc_tiling_on_sc=False),
        scratch_types=[
            pltpu.VMEM((rows_w, GRP), jnp.int32),
            pltpu.VMEM((NBUF, KG, GRP, D), jnp.float32),
            pltpu.VMEM((rows_w, GRP), jnp.int32),
        ]
        + [pltpu.SemaphoreType.DMA] * (2 * NBUF),
    )
    def k(tab_hbm, x_hbm, out_hbm, mask_hbm, idx_v, rows_v, mask_v, *sems):
        gat_sems = sems[:NBUF]
        out_sems = sems[NBUF:]
        wid = lax.axis_index("s") * NC + lax.axis_index("c")
        row0 = wid * rows_w

        pltpu.sync_copy(x_hbm.at[pl.ds(row0, rows_w)], idx_v)

        def fire_gathers(g, b):
            for j in range(KG):
                pltpu.async_copy(
                    tab_hbm.at[idx_v.at[g * KG + j]],
                    rows_v.at[b].at[j], gat_sems[b])

        def wait_gathers(b):
            for j in range(KG):
                pltpu.make_async_copy(
                    tab_hbm.at[idx_v.at[j]],
                    rows_v.at[b].at[j], gat_sems[b]).wait()

        def fire_out(g, b):
            pltpu.async_copy(
                rows_v.at[b], out_hbm.at[pl.ds(row0 + g * KG, KG)],
                out_sems[b])

        def wait_out(b):
            pltpu.make_async_copy(
                rows_v.at[b], out_hbm.at[pl.ds(row0, KG)],
                out_sems[b]).wait()

        def mask_chunk(g):
            for j in range(KG):
                for v in range(GRP // 16):
                    iv = idx_v[g * KG + j, pl.ds(v * 16, 16)]
                    mask_v[g * KG + j, pl.ds(v * 16, 16)] = jnp.where(
                        iv > 0, jnp.int32(1), jnp.int32(0))

        def round_body(i, carry):
            for b in range(NBUF):
                g = i * NBUF + b
                @pl.when(i > 0)
                def _(b=b):
                    wait_out(b)
                fire_gathers(g, b)
                pb = b - 1 if b > 0 else NBUF - 1
                pg = g - 1

                @pl.when((i > 0) | (b > 0))
                def _(pb=pb, pg=pg):
                    wait_gathers(pb)
                    fire_out(pg, pb)
                mask_chunk(g)
            return carry

        lax.fori_loop(0, nblk // NBUF, round_body, 0, unroll=False)

        last = nblk - 1
        lb = last % NBUF
        wait_gathers(lb)
        fire_out(last, lb)
        for b in range(NBUF):
            wait_out(b)

        pltpu.sync_copy(mask_v, mask_hbm.at[pl.ds(row0, rows_w)])

    return k(ntable, x2d)


def kernel(x, table, ln_weight, ln_bias):
    B, S = x.shape
    V, D = table.shape
    N = B * S
    assert N % (NW * BLK) == 0

    ntable = _normalize_table(table, ln_weight, ln_bias)
    x2d = x.astype(jnp.int32).reshape(N // GRP, GRP)
    out, mask_i32 = _gather_mask_sc(ntable, x2d)
    normed = out.reshape(B, S, D)
    mask = (mask_i32 != 0).reshape(B, S)
    return (normed, mask)
